# baseline (speedup 1.0000x reference)
import functools

import jax
import jax.numpy as jnp
from jax import lax
from jax.experimental import pallas as pl
from jax.experimental.pallas import tpu as pltpu
from jax.experimental.pallas import tpu_sc as plsc

_N = 10000
_E = 320000
_D = 128
_NC, _NS = 2, 16
_NW = _NC * _NS
_CH = 128
_TCH = 2560
_EPAD = _TCH * _CH
_PH = 16
_N0 = 32
_N1 = 128
_DEG_CH = _TCH // _NW
_ACC_ROWS = 10240
_ZCH = _ACC_ROWS // _NS // _CH
_RPT = _ACC_ROWS // _NS
_DEG_W = 128

_ROW_BLK = 2000


def _sc_mesh():
    return plsc.VectorSubcoreMesh(core_axis_name="c", subcore_axis_name="s")


def _deg_body(dst_hbm, out_hbm, dst_v, buf, acc):
    c = lax.axis_index("c")
    s = lax.axis_index("s")
    wid = s * _NC + c
    zeros16 = jnp.zeros((16,), jnp.float32)
    ones16 = jnp.ones((16,), jnp.float32)

    @pl.loop(0, _CH)
    def _(i):
        for l in range(_DEG_W // 16):
            buf[i, pl.ds(l * 16, 16)] = zeros16

    @pl.loop(0, _ZCH)
    def _(t):
        pltpu.sync_copy(buf, acc.at[pl.ds(s * _RPT + t * _CH, _CH)])

    @pl.loop(0, _CH)
    def _(i):
        for l in range(_DEG_W // 16):
            buf[i, pl.ds(l * 16, 16)] = ones16

    pltpu.sync_copy(dst_hbm.at[pl.ds(wid * _DEG_CH, _DEG_CH)], dst_v)
    plsc.subcore_barrier()

    @pl.loop(0, _DEG_CH)
    def _(j):
        pltpu.sync_copy(buf, acc.at[dst_v.at[j]], add=True)

    plsc.subcore_barrier()
    pltpu.sync_copy(acc.at[pl.ds(s * _RPT, _RPT)], out_hbm.at[c, pl.ds(s * _RPT, _RPT)])


def _deg_kernel(dst_slab):
    k = pl.kernel(
        _deg_body,
        out_type=jax.ShapeDtypeStruct((_NC, _ACC_ROWS, _DEG_W), jnp.float32),
        mesh=_sc_mesh(),
        scratch_types=[
            pltpu.VMEM((_DEG_CH, _CH), jnp.int32),
            pltpu.VMEM((_CH, _DEG_W), jnp.float32),
            pltpu.VMEM_SHARED((_ACC_ROWS, _DEG_W), jnp.float32),
        ],
    )
    return k(dst_slab)


def _agg_body(hs_hbm, src_hbm, dst_hbm, out_hbm,
              src_v, dst_v, buf0, buf1, acc, semg0, semg1):
    c = lax.axis_index("c")
    s = lax.axis_index("s")
    zeros16 = jnp.zeros((16,), jnp.float32)

    @pl.loop(0, _CH)
    def _(i):
        for l in range(_D // 16):
            buf0[i, pl.ds(l * 16, 16)] = zeros16

    @pl.loop(0, _ZCH)
    def _(t):
        pltpu.sync_copy(
            buf0, acc.at[pl.ds(s * (_ACC_ROWS // _NS) + t * _CH, _CH)])

    plsc.subcore_barrier()

    nph = jnp.where(c == 0, _N0 // _PH, _N1 // _PH)
    base = jnp.where(c == 0, s * _N0, _NS * _N0 + s * _N1)

    @pl.loop(0, nph)
    def _(ph):
        cb = base + ph * _PH
        pltpu.sync_copy(src_hbm.at[pl.ds(cb, _PH)], src_v)
        pltpu.sync_copy(dst_hbm.at[pl.ds(cb, _PH)], dst_v)
        pltpu.async_copy(hs_hbm.at[src_v.at[0]], buf0, semg0)

        @pl.loop(0, _PH - 2, step=2)
        def _(j):
            pltpu.async_copy(hs_hbm.at[src_v.at[j + 1]], buf1, semg1)
            pltpu.make_async_copy(hs_hbm.at[src_v.at[j]], buf0, semg0).wait()
            pltpu.sync_copy(buf0, acc.at[dst_v.at[j]], add=True)
            pltpu.async_copy(hs_hbm.at[src_v.at[j + 2]], buf0, semg0)
            pltpu.make_async_copy(hs_hbm.at[src_v.at[j + 1]], buf1, semg1).wait()
            pltpu.sync_copy(buf1, acc.at[dst_v.at[j + 1]], add=True)

        pltpu.async_copy(hs_hbm.at[src_v.at[_PH - 1]], buf1, semg1)
        pltpu.make_async_copy(hs_hbm.at[src_v.at[_PH - 2]], buf0, semg0).wait()
        pltpu.sync_copy(buf0, acc.at[dst_v.at[_PH - 2]], add=True)
        pltpu.make_async_copy(hs_hbm.at[src_v.at[_PH - 1]], buf1, semg1).wait()
        pltpu.sync_copy(buf1, acc.at[dst_v.at[_PH - 1]], add=True)

    plsc.subcore_barrier()
    pltpu.sync_copy(acc.at[pl.ds(s * _RPT, _RPT)], out_hbm.at[c, pl.ds(s * _RPT, _RPT)])


def _agg_kernel(hs, src_slab, dst_slab):
    k = pl.kernel(
        _agg_body,
        out_type=jax.ShapeDtypeStruct((_NC, _ACC_ROWS, _D), jnp.float32),
        mesh=_sc_mesh(),
        scratch_types=[
            pltpu.VMEM((_PH, _CH), jnp.int32),
            pltpu.VMEM((_PH, _CH), jnp.int32),
            pltpu.VMEM((_CH, _D), jnp.float32),
            pltpu.VMEM((_CH, _D), jnp.float32),
            pltpu.VMEM_SHARED((_ACC_ROWS, _D), jnp.float32),
            pltpu.SemaphoreType.DMA,
            pltpu.SemaphoreType.DMA,
        ],
    )
    return k(hs, src_slab, dst_slab)


def _dinv_blk(deg2_ref):
    deg = deg2_ref[0, :, 0:1] + deg2_ref[1, :, 0:1] + 1.0
    return lax.rsqrt(deg)


def _pre_body(x_ref, w_ref, deg2_ref, o_ref):
    dinv = _dinv_blk(deg2_ref)
    o_ref[...] = dinv * jnp.dot(x_ref[...], w_ref[...],
                                preferred_element_type=jnp.float32)


def _mid_body(s_ref, hs_ref, deg2_ref, w_ref, b_ref, o_ref):
    dinv = _dinv_blk(deg2_ref)
    pre = s_ref[0] + s_ref[1] + hs_ref[...]
    xn = jnp.maximum(dinv * pre + b_ref[...], 0.0)
    o_ref[...] = dinv * jnp.dot(xn, w_ref[...],
                                preferred_element_type=jnp.float32)


def _post_body(s_ref, hs_ref, deg2_ref, b_ref, o_ref):
    dinv = _dinv_blk(deg2_ref)
    o_ref[...] = dinv * (s_ref[0] + s_ref[1] + hs_ref[...]) + b_ref[...]


_GRID = (_N // _ROW_BLK,)
_SPEC_ROWS = pl.BlockSpec((_ROW_BLK, _D), lambda i: (i, 0))
_SPEC_S = pl.BlockSpec((_NC, _ROW_BLK, _D), lambda i: (0, i, 0))
_SPEC_DEG = pl.BlockSpec((_NC, _ROW_BLK, _DEG_W), lambda i: (0, i, 0))
_SPEC_W = pl.BlockSpec((_D, _D), lambda i: (0, 0))
_SPEC_B = pl.BlockSpec((1, _D), lambda i: (0, 0))
_OUT_SD = jax.ShapeDtypeStruct((_N, _D), jnp.float32)


def _pre_kernel(x, W, deg2):
    return pl.pallas_call(
        _pre_body, grid=_GRID,
        in_specs=[_SPEC_ROWS, _SPEC_W, _SPEC_DEG],
        out_specs=_SPEC_ROWS, out_shape=_OUT_SD,
    )(x, W, deg2)


def _mid_kernel(S, hs, deg2, Wn, b):
    return pl.pallas_call(
        _mid_body, grid=_GRID,
        in_specs=[_SPEC_S, _SPEC_ROWS, _SPEC_DEG, _SPEC_W, _SPEC_B],
        out_specs=_SPEC_ROWS, out_shape=_OUT_SD,
    )(S, hs, deg2, Wn, b)


def _post_kernel(S, hs, deg2, b):
    return pl.pallas_call(
        _post_body, grid=_GRID,
        in_specs=[_SPEC_S, _SPEC_ROWS, _SPEC_DEG, _SPEC_B],
        out_specs=_SPEC_ROWS, out_shape=_OUT_SD,
    )(S, hs, deg2, b)


def kernel(x, edge_index, W0, b0, W1, b1, W2, b2):
    pad = _EPAD - _E
    src = jnp.concatenate([edge_index[0], jnp.zeros((pad,), jnp.int32)])
    dst = jnp.concatenate([edge_index[1], jnp.full((pad,), _N, jnp.int32)])
    src_slab = src.reshape(_TCH, _CH)
    dst_slab = dst.reshape(_TCH, _CH)

    deg2 = _deg_kernel(dst_slab)

    hs0 = _pre_kernel(x, W0, deg2)
    S0 = _agg_kernel(hs0, src_slab, dst_slab)
    hs1 = _mid_kernel(S0, hs0, deg2, W1, b0.reshape(1, _D))
    S1 = _agg_kernel(hs1, src_slab, dst_slab)
    hs2 = _mid_kernel(S1, hs1, deg2, W2, b1.reshape(1, _D))
    S2 = _agg_kernel(hs2, src_slab, dst_slab)
    return _post_kernel(S2, hs2, deg2, b2.reshape(1, _D))


# speedup vs baseline: 1.1418x; 1.1418x over previous
import functools

import jax
import jax.numpy as jnp
from jax import lax
from jax.experimental import pallas as pl
from jax.experimental.pallas import tpu as pltpu
from jax.experimental.pallas import tpu_sc as plsc

_N = 10000
_E = 320000
_D = 128
_NC, _NS = 2, 16
_NW = _NC * _NS
_CH = 128
_TCH = 2560
_EPAD = _TCH * _CH
_PH = 16
_N0 = 128
_N1 = 32
_DEG_CH = _TCH // _NW
_ACC_ROWS = 10240
_ZCH = _ACC_ROWS // _NS // _CH
_RPT = _ACC_ROWS // _NS
_DEG_W = 128

_ROW_BLK = 2000


def _sc_mesh():
    return plsc.VectorSubcoreMesh(core_axis_name="c", subcore_axis_name="s")


def _deg_body(dst_hbm, out_hbm, dst_v, buf, acc):
    c = lax.axis_index("c")
    s = lax.axis_index("s")
    wid = s * _NC + c
    zeros16 = jnp.zeros((16,), jnp.float32)
    ones16 = jnp.ones((16,), jnp.float32)

    @pl.loop(0, _CH)
    def _(i):
        for l in range(_DEG_W // 16):
            buf[i, pl.ds(l * 16, 16)] = zeros16

    @pl.loop(0, _ZCH)
    def _(t):
        pltpu.sync_copy(buf, acc.at[pl.ds(s * _RPT + t * _CH, _CH)])

    @pl.loop(0, _CH)
    def _(i):
        for l in range(_DEG_W // 16):
            buf[i, pl.ds(l * 16, 16)] = ones16

    pltpu.sync_copy(dst_hbm.at[pl.ds(wid * _DEG_CH, _DEG_CH)], dst_v)
    plsc.subcore_barrier()

    @pl.loop(0, _DEG_CH)
    def _(j):
        pltpu.sync_copy(buf, acc.at[dst_v.at[j]], add=True)

    plsc.subcore_barrier()
    pltpu.sync_copy(acc.at[pl.ds(s * _RPT, _RPT)], out_hbm.at[c, pl.ds(s * _RPT, _RPT)])


def _deg_kernel(dst_slab):
    k = pl.kernel(
        _deg_body,
        out_type=jax.ShapeDtypeStruct((_NC, _ACC_ROWS, _DEG_W), jnp.float32),
        mesh=_sc_mesh(),
        scratch_types=[
            pltpu.VMEM((_DEG_CH, _CH), jnp.int32),
            pltpu.VMEM((_CH, _DEG_W), jnp.float32),
            pltpu.VMEM_SHARED((_ACC_ROWS, _DEG_W), jnp.float32),
        ],
    )
    return k(dst_slab)


def _agg_body(hs_hbm, src_hbm, dst_hbm, out_hbm,
              src_v, dst_v, buf0, buf1, acc, semg0, semg1):
    c = lax.axis_index("c")
    s = lax.axis_index("s")
    zeros16 = jnp.zeros((16,), jnp.float32)

    @pl.loop(0, _CH)
    def _(i):
        for l in range(_D // 16):
            buf0[i, pl.ds(l * 16, 16)] = zeros16

    @pl.loop(0, _ZCH)
    def _(t):
        pltpu.sync_copy(
            buf0, acc.at[pl.ds(s * (_ACC_ROWS // _NS) + t * _CH, _CH)])

    plsc.subcore_barrier()

    nph = jnp.where(c == 0, _N0 // _PH, _N1 // _PH)
    base = jnp.where(c == 0, s * _N0, _NS * _N0 + s * _N1)

    @pl.loop(0, nph)
    def _(ph):
        cb = base + ph * _PH
        pltpu.sync_copy(src_hbm.at[pl.ds(cb, _PH)], src_v)
        pltpu.sync_copy(dst_hbm.at[pl.ds(cb, _PH)], dst_v)
        pltpu.async_copy(hs_hbm.at[src_v.at[0]], buf0, semg0)

        @pl.loop(0, _PH - 2, step=2)
        def _(j):
            pltpu.async_copy(hs_hbm.at[src_v.at[j + 1]], buf1, semg1)
            pltpu.make_async_copy(hs_hbm.at[src_v.at[j]], buf0, semg0).wait()
            pltpu.sync_copy(buf0, acc.at[dst_v.at[j]], add=True)
            pltpu.async_copy(hs_hbm.at[src_v.at[j + 2]], buf0, semg0)
            pltpu.make_async_copy(hs_hbm.at[src_v.at[j + 1]], buf1, semg1).wait()
            pltpu.sync_copy(buf1, acc.at[dst_v.at[j + 1]], add=True)

        pltpu.async_copy(hs_hbm.at[src_v.at[_PH - 1]], buf1, semg1)
        pltpu.make_async_copy(hs_hbm.at[src_v.at[_PH - 2]], buf0, semg0).wait()
        pltpu.sync_copy(buf0, acc.at[dst_v.at[_PH - 2]], add=True)
        pltpu.make_async_copy(hs_hbm.at[src_v.at[_PH - 1]], buf1, semg1).wait()
        pltpu.sync_copy(buf1, acc.at[dst_v.at[_PH - 1]], add=True)

    plsc.subcore_barrier()
    pltpu.sync_copy(acc.at[pl.ds(s * _RPT, _RPT)], out_hbm.at[c, pl.ds(s * _RPT, _RPT)])


def _agg_kernel(hs, src_slab, dst_slab):
    k = pl.kernel(
        _agg_body,
        out_type=jax.ShapeDtypeStruct((_NC, _ACC_ROWS, _D), jnp.float32),
        mesh=_sc_mesh(),
        scratch_types=[
            pltpu.VMEM((_PH, _CH), jnp.int32),
            pltpu.VMEM((_PH, _CH), jnp.int32),
            pltpu.VMEM((_CH, _D), jnp.float32),
            pltpu.VMEM((_CH, _D), jnp.float32),
            pltpu.VMEM_SHARED((_ACC_ROWS, _D), jnp.float32),
            pltpu.SemaphoreType.DMA,
            pltpu.SemaphoreType.DMA,
        ],
    )
    return k(hs, src_slab, dst_slab)


def _dinv_blk(deg2_ref):
    deg = deg2_ref[0, :, 0:1] + deg2_ref[1, :, 0:1] + 1.0
    return lax.rsqrt(deg)


def _pre_body(x_ref, w_ref, deg2_ref, o_ref):
    dinv = _dinv_blk(deg2_ref)
    o_ref[...] = dinv * jnp.dot(x_ref[...], w_ref[...],
                                preferred_element_type=jnp.float32)


def _mid_body(s_ref, hs_ref, deg2_ref, w_ref, b_ref, o_ref):
    dinv = _dinv_blk(deg2_ref)
    pre = s_ref[0] + s_ref[1] + hs_ref[...]
    xn = jnp.maximum(dinv * pre + b_ref[...], 0.0)
    o_ref[...] = dinv * jnp.dot(xn, w_ref[...],
                                preferred_element_type=jnp.float32)


def _post_body(s_ref, hs_ref, deg2_ref, b_ref, o_ref):
    dinv = _dinv_blk(deg2_ref)
    o_ref[...] = dinv * (s_ref[0] + s_ref[1] + hs_ref[...]) + b_ref[...]


_GRID = (_N // _ROW_BLK,)
_SPEC_ROWS = pl.BlockSpec((_ROW_BLK, _D), lambda i: (i, 0))
_SPEC_S = pl.BlockSpec((_NC, _ROW_BLK, _D), lambda i: (0, i, 0))
_SPEC_DEG = pl.BlockSpec((_NC, _ROW_BLK, _DEG_W), lambda i: (0, i, 0))
_SPEC_W = pl.BlockSpec((_D, _D), lambda i: (0, 0))
_SPEC_B = pl.BlockSpec((1, _D), lambda i: (0, 0))
_OUT_SD = jax.ShapeDtypeStruct((_N, _D), jnp.float32)


def _pre_kernel(x, W, deg2):
    return pl.pallas_call(
        _pre_body, grid=_GRID,
        in_specs=[_SPEC_ROWS, _SPEC_W, _SPEC_DEG],
        out_specs=_SPEC_ROWS, out_shape=_OUT_SD,
    )(x, W, deg2)


def _mid_kernel(S, hs, deg2, Wn, b):
    return pl.pallas_call(
        _mid_body, grid=_GRID,
        in_specs=[_SPEC_S, _SPEC_ROWS, _SPEC_DEG, _SPEC_W, _SPEC_B],
        out_specs=_SPEC_ROWS, out_shape=_OUT_SD,
    )(S, hs, deg2, Wn, b)


def _post_kernel(S, hs, deg2, b):
    return pl.pallas_call(
        _post_body, grid=_GRID,
        in_specs=[_SPEC_S, _SPEC_ROWS, _SPEC_DEG, _SPEC_B],
        out_specs=_SPEC_ROWS, out_shape=_OUT_SD,
    )(S, hs, deg2, b)


def kernel(x, edge_index, W0, b0, W1, b1, W2, b2):
    pad = _EPAD - _E
    src = jnp.concatenate([edge_index[0], jnp.zeros((pad,), jnp.int32)])
    dst = jnp.concatenate([edge_index[1], jnp.full((pad,), _N, jnp.int32)])
    src_slab = src.reshape(_TCH, _CH)
    dst_slab = dst.reshape(_TCH, _CH)

    deg2 = _deg_kernel(dst_slab)

    hs0 = _pre_kernel(x, W0, deg2)
    S0 = _agg_kernel(hs0, src_slab, dst_slab)
    hs1 = _mid_kernel(S0, hs0, deg2, W1, b0.reshape(1, _D))
    S1 = _agg_kernel(hs1, src_slab, dst_slab)
    hs2 = _mid_kernel(S1, hs1, deg2, W2, b1.reshape(1, _D))
    S2 = _agg_kernel(hs2, src_slab, dst_slab)
    return _post_kernel(S2, hs2, deg2, b2.reshape(1, _D))


# speedup vs baseline: 1.1431x; 1.0011x over previous
import functools

import jax
import jax.numpy as jnp
from jax import lax
from jax.experimental import pallas as pl
from jax.experimental.pallas import tpu as pltpu
from jax.experimental.pallas import tpu_sc as plsc

_N = 10000
_E = 320000
_D = 128
_NC, _NS = 2, 16
_NW = _NC * _NS
_CH = 128
_TCH = 2560
_EPAD = _TCH * _CH
_PH = 32
_N0 = 128
_N1 = 32
_DEG_CH = _TCH // _NW
_ACC_ROWS = 10240
_ZCH = _ACC_ROWS // _NS // _CH
_RPT = _ACC_ROWS // _NS
_DEG_W = 128

_ROW_BLK = 2000


def _sc_mesh():
    return plsc.VectorSubcoreMesh(core_axis_name="c", subcore_axis_name="s")


def _deg_body(dst_hbm, out_hbm, dst_v, buf, acc):
    c = lax.axis_index("c")
    s = lax.axis_index("s")
    wid = s * _NC + c
    zeros16 = jnp.zeros((16,), jnp.float32)
    ones16 = jnp.ones((16,), jnp.float32)

    @pl.loop(0, _CH)
    def _(i):
        for l in range(_DEG_W // 16):
            buf[i, pl.ds(l * 16, 16)] = zeros16

    @pl.loop(0, _ZCH)
    def _(t):
        pltpu.sync_copy(buf, acc.at[pl.ds(s * _RPT + t * _CH, _CH)])

    @pl.loop(0, _CH)
    def _(i):
        for l in range(_DEG_W // 16):
            buf[i, pl.ds(l * 16, 16)] = ones16

    pltpu.sync_copy(dst_hbm.at[pl.ds(wid * _DEG_CH, _DEG_CH)], dst_v)
    plsc.subcore_barrier()

    @pl.loop(0, _DEG_CH)
    def _(j):
        pltpu.sync_copy(buf, acc.at[dst_v.at[j]], add=True)

    plsc.subcore_barrier()
    pltpu.sync_copy(acc.at[pl.ds(s * _RPT, _RPT)], out_hbm.at[c, pl.ds(s * _RPT, _RPT)])


def _deg_kernel(dst_slab):
    k = pl.kernel(
        _deg_body,
        out_type=jax.ShapeDtypeStruct((_NC, _ACC_ROWS, _DEG_W), jnp.float32),
        mesh=_sc_mesh(),
        scratch_types=[
            pltpu.VMEM((_DEG_CH, _CH), jnp.int32),
            pltpu.VMEM((_CH, _DEG_W), jnp.float32),
            pltpu.VMEM_SHARED((_ACC_ROWS, _DEG_W), jnp.float32),
        ],
    )
    return k(dst_slab)


def _agg_body(hs_hbm, src_hbm, dst_hbm, out_hbm,
              src_v, dst_v, buf0, buf1, acc, semg0, semg1):
    c = lax.axis_index("c")
    s = lax.axis_index("s")
    zeros16 = jnp.zeros((16,), jnp.float32)

    @pl.loop(0, _CH)
    def _(i):
        for l in range(_D // 16):
            buf0[i, pl.ds(l * 16, 16)] = zeros16

    @pl.loop(0, _ZCH)
    def _(t):
        pltpu.sync_copy(
            buf0, acc.at[pl.ds(s * (_ACC_ROWS // _NS) + t * _CH, _CH)])

    plsc.subcore_barrier()

    nph = jnp.where(c == 0, _N0 // _PH, _N1 // _PH)
    base = jnp.where(c == 0, s * _N0, _NS * _N0 + s * _N1)

    @pl.loop(0, nph)
    def _(ph):
        cb = base + ph * _PH
        pltpu.sync_copy(src_hbm.at[pl.ds(cb, _PH)], src_v)
        pltpu.sync_copy(dst_hbm.at[pl.ds(cb, _PH)], dst_v)
        pltpu.async_copy(hs_hbm.at[src_v.at[0]], buf0, semg0)

        @pl.loop(0, _PH - 2, step=2)
        def _(j):
            pltpu.async_copy(hs_hbm.at[src_v.at[j + 1]], buf1, semg1)
            pltpu.make_async_copy(hs_hbm.at[src_v.at[j]], buf0, semg0).wait()
            pltpu.sync_copy(buf0, acc.at[dst_v.at[j]], add=True)
            pltpu.async_copy(hs_hbm.at[src_v.at[j + 2]], buf0, semg0)
            pltpu.make_async_copy(hs_hbm.at[src_v.at[j + 1]], buf1, semg1).wait()
            pltpu.sync_copy(buf1, acc.at[dst_v.at[j + 1]], add=True)

        pltpu.async_copy(hs_hbm.at[src_v.at[_PH - 1]], buf1, semg1)
        pltpu.make_async_copy(hs_hbm.at[src_v.at[_PH - 2]], buf0, semg0).wait()
        pltpu.sync_copy(buf0, acc.at[dst_v.at[_PH - 2]], add=True)
        pltpu.make_async_copy(hs_hbm.at[src_v.at[_PH - 1]], buf1, semg1).wait()
        pltpu.sync_copy(buf1, acc.at[dst_v.at[_PH - 1]], add=True)

    plsc.subcore_barrier()
    pltpu.sync_copy(acc.at[pl.ds(s * _RPT, _RPT)], out_hbm.at[c, pl.ds(s * _RPT, _RPT)])


def _agg_kernel(hs, src_slab, dst_slab):
    k = pl.kernel(
        _agg_body,
        out_type=jax.ShapeDtypeStruct((_NC, _ACC_ROWS, _D), jnp.float32),
        mesh=_sc_mesh(),
        scratch_types=[
            pltpu.VMEM((_PH, _CH), jnp.int32),
            pltpu.VMEM((_PH, _CH), jnp.int32),
            pltpu.VMEM((_CH, _D), jnp.float32),
            pltpu.VMEM((_CH, _D), jnp.float32),
            pltpu.VMEM_SHARED((_ACC_ROWS, _D), jnp.float32),
            pltpu.SemaphoreType.DMA,
            pltpu.SemaphoreType.DMA,
        ],
    )
    return k(hs, src_slab, dst_slab)


def _dinv_blk(deg2_ref):
    deg = deg2_ref[0, :, 0:1] + deg2_ref[1, :, 0:1] + 1.0
    return lax.rsqrt(deg)


def _pre_body(x_ref, w_ref, deg2_ref, o_ref):
    dinv = _dinv_blk(deg2_ref)
    o_ref[...] = dinv * jnp.dot(x_ref[...], w_ref[...],
                                preferred_element_type=jnp.float32)


def _mid_body(s_ref, hs_ref, deg2_ref, w_ref, b_ref, o_ref):
    dinv = _dinv_blk(deg2_ref)
    pre = s_ref[0] + s_ref[1] + hs_ref[...]
    xn = jnp.maximum(dinv * pre + b_ref[...], 0.0)
    o_ref[...] = dinv * jnp.dot(xn, w_ref[...],
                                preferred_element_type=jnp.float32)


def _post_body(s_ref, hs_ref, deg2_ref, b_ref, o_ref):
    dinv = _dinv_blk(deg2_ref)
    o_ref[...] = dinv * (s_ref[0] + s_ref[1] + hs_ref[...]) + b_ref[...]


_GRID = (_N // _ROW_BLK,)
_SPEC_ROWS = pl.BlockSpec((_ROW_BLK, _D), lambda i: (i, 0))
_SPEC_S = pl.BlockSpec((_NC, _ROW_BLK, _D), lambda i: (0, i, 0))
_SPEC_DEG = pl.BlockSpec((_NC, _ROW_BLK, _DEG_W), lambda i: (0, i, 0))
_SPEC_W = pl.BlockSpec((_D, _D), lambda i: (0, 0))
_SPEC_B = pl.BlockSpec((1, _D), lambda i: (0, 0))
_OUT_SD = jax.ShapeDtypeStruct((_N, _D), jnp.float32)


def _pre_kernel(x, W, deg2):
    return pl.pallas_call(
        _pre_body, grid=_GRID,
        in_specs=[_SPEC_ROWS, _SPEC_W, _SPEC_DEG],
        out_specs=_SPEC_ROWS, out_shape=_OUT_SD,
    )(x, W, deg2)


def _mid_kernel(S, hs, deg2, Wn, b):
    return pl.pallas_call(
        _mid_body, grid=_GRID,
        in_specs=[_SPEC_S, _SPEC_ROWS, _SPEC_DEG, _SPEC_W, _SPEC_B],
        out_specs=_SPEC_ROWS, out_shape=_OUT_SD,
    )(S, hs, deg2, Wn, b)


def _post_kernel(S, hs, deg2, b):
    return pl.pallas_call(
        _post_body, grid=_GRID,
        in_specs=[_SPEC_S, _SPEC_ROWS, _SPEC_DEG, _SPEC_B],
        out_specs=_SPEC_ROWS, out_shape=_OUT_SD,
    )(S, hs, deg2, b)


def kernel(x, edge_index, W0, b0, W1, b1, W2, b2):
    pad = _EPAD - _E
    src = jnp.concatenate([edge_index[0], jnp.zeros((pad,), jnp.int32)])
    dst = jnp.concatenate([edge_index[1], jnp.full((pad,), _N, jnp.int32)])
    src_slab = src.reshape(_TCH, _CH)
    dst_slab = dst.reshape(_TCH, _CH)

    deg2 = _deg_kernel(dst_slab)

    hs0 = _pre_kernel(x, W0, deg2)
    S0 = _agg_kernel(hs0, src_slab, dst_slab)
    hs1 = _mid_kernel(S0, hs0, deg2, W1, b0.reshape(1, _D))
    S1 = _agg_kernel(hs1, src_slab, dst_slab)
    hs2 = _mid_kernel(S1, hs1, deg2, W2, b1.reshape(1, _D))
    S2 = _agg_kernel(hs2, src_slab, dst_slab)
    return _post_kernel(S2, hs2, deg2, b2.reshape(1, _D))


# speedup vs baseline: 1.8515x; 1.6198x over previous
import functools

import jax
import jax.numpy as jnp
from jax import lax
from jax.experimental import pallas as pl
from jax.experimental.pallas import tpu as pltpu
from jax.experimental.pallas import tpu_sc as plsc

_N = 10000
_E = 320000
_D = 128
_NC, _NS = 2, 16
_NW = _NC * _NS
_CH = 128
_TCH = 2560
_EPAD = _TCH * _CH
_PH = 32
_N0 = 112
_N1 = 48
_DEG_CH = _TCH // _NW
_ACC_ROWS = 10240
_ZCH = _ACC_ROWS // _NS // _CH
_RPT = _ACC_ROWS // _NS
_DEG_W = 128

_ROW_BLK = 2000


def _sc_mesh():
    return plsc.VectorSubcoreMesh(core_axis_name="c", subcore_axis_name="s")


def _deg_body(dst_hbm, out_hbm, dst_v, buf, acc):
    c = lax.axis_index("c")
    s = lax.axis_index("s")
    wid = s * _NC + c
    zeros16 = jnp.zeros((16,), jnp.float32)
    ones16 = jnp.ones((16,), jnp.float32)

    @pl.loop(0, _CH)
    def _(i):
        for l in range(_DEG_W // 16):
            buf[i, pl.ds(l * 16, 16)] = zeros16

    @pl.loop(0, _ZCH)
    def _(t):
        pltpu.sync_copy(buf, acc.at[pl.ds(s * _RPT + t * _CH, _CH)])

    @pl.loop(0, _CH)
    def _(i):
        for l in range(_DEG_W // 16):
            buf[i, pl.ds(l * 16, 16)] = ones16

    pltpu.sync_copy(dst_hbm.at[pl.ds(wid * _DEG_CH, _DEG_CH)], dst_v)
    plsc.subcore_barrier()

    @pl.loop(0, _DEG_CH)
    def _(j):
        pltpu.sync_copy(buf, acc.at[dst_v.at[j]], add=True)

    plsc.subcore_barrier()
    pltpu.sync_copy(acc.at[pl.ds(s * _RPT, _RPT)], out_hbm.at[c, pl.ds(s * _RPT, _RPT)])


def _deg_kernel(dst_slab):
    k = pl.kernel(
        _deg_body,
        out_type=jax.ShapeDtypeStruct((_NC, _ACC_ROWS, _DEG_W), jnp.float32),
        mesh=_sc_mesh(),
        scratch_types=[
            pltpu.VMEM((_DEG_CH, _CH), jnp.int32),
            pltpu.VMEM((_CH, _DEG_W), jnp.float32),
            pltpu.VMEM_SHARED((_ACC_ROWS, _DEG_W), jnp.float32),
        ],
    )
    return k(dst_slab)


def _agg_body(hs_hbm, src_hbm, dst_hbm, out_hbm,
              src_v, dst_v, buf0, buf1, acc, semg0, semg1):
    c = lax.axis_index("c")
    s = lax.axis_index("s")
    zeros16 = jnp.zeros((16,), jnp.float32)

    @pl.loop(0, _CH)
    def _(i):
        for l in range(_D // 16):
            buf0[i, pl.ds(l * 16, 16)] = zeros16

    @pl.loop(0, _ZCH)
    def _(t):
        pltpu.sync_copy(
            buf0, acc.at[pl.ds(s * (_ACC_ROWS // _NS) + t * _CH, _CH)])

    plsc.subcore_barrier()

    nph = jnp.where(c == 0, _N0 // _PH, _N1 // _PH)
    base = jnp.where(c == 0, s * _N0, _NS * _N0 + s * _N1)

    @pl.loop(0, nph)
    def _(ph):
        cb = base + ph * _PH
        pltpu.sync_copy(src_hbm.at[pl.ds(cb, _PH)], src_v)
        pltpu.sync_copy(dst_hbm.at[pl.ds(cb, _PH)], dst_v)
        pltpu.async_copy(hs_hbm.at[src_v.at[0]], buf0, semg0)

        @pl.loop(0, _PH - 2, step=2)
        def _(j):
            pltpu.async_copy(hs_hbm.at[src_v.at[j + 1]], buf1, semg1)
            pltpu.make_async_copy(hs_hbm.at[src_v.at[j]], buf0, semg0).wait()
            pltpu.sync_copy(buf0, acc.at[dst_v.at[j]], add=True)
            pltpu.async_copy(hs_hbm.at[src_v.at[j + 2]], buf0, semg0)
            pltpu.make_async_copy(hs_hbm.at[src_v.at[j + 1]], buf1, semg1).wait()
            pltpu.sync_copy(buf1, acc.at[dst_v.at[j + 1]], add=True)

        pltpu.async_copy(hs_hbm.at[src_v.at[_PH - 1]], buf1, semg1)
        pltpu.make_async_copy(hs_hbm.at[src_v.at[_PH - 2]], buf0, semg0).wait()
        pltpu.sync_copy(buf0, acc.at[dst_v.at[_PH - 2]], add=True)
        pltpu.make_async_copy(hs_hbm.at[src_v.at[_PH - 1]], buf1, semg1).wait()
        pltpu.sync_copy(buf1, acc.at[dst_v.at[_PH - 1]], add=True)

    plsc.subcore_barrier()
    pltpu.sync_copy(acc.at[pl.ds(s * _RPT, _RPT)], out_hbm.at[c, pl.ds(s * _RPT, _RPT)])


def _agg_kernel(hs, src_slab, dst_slab):
    k = pl.kernel(
        _agg_body,
        out_type=jax.ShapeDtypeStruct((_NC, _ACC_ROWS, _D), jnp.float32),
        mesh=_sc_mesh(),
        scratch_types=[
            pltpu.VMEM((_PH, _CH), jnp.int32),
            pltpu.VMEM((_PH, _CH), jnp.int32),
            pltpu.VMEM((_CH, _D), jnp.float32),
            pltpu.VMEM((_CH, _D), jnp.float32),
            pltpu.VMEM_SHARED((_ACC_ROWS, _D), jnp.float32),
            pltpu.SemaphoreType.DMA,
            pltpu.SemaphoreType.DMA,
        ],
    )
    return k(hs, src_slab, dst_slab)


def _dinv_blk(deg2_ref):
    deg = deg2_ref[0, :, 0:1] + deg2_ref[1, :, 0:1] + 1.0
    return lax.rsqrt(deg)


def _pre_body(x_ref, w_ref, deg2_ref, o_ref):
    dinv = _dinv_blk(deg2_ref)
    o_ref[...] = dinv * jnp.dot(x_ref[...], w_ref[...],
                                preferred_element_type=jnp.float32)


def _mid_body(s_ref, hs_ref, deg2_ref, w_ref, b_ref, o_ref):
    dinv = _dinv_blk(deg2_ref)
    pre = s_ref[0] + s_ref[1] + hs_ref[...]
    xn = jnp.maximum(dinv * pre + b_ref[...], 0.0)
    o_ref[...] = dinv * jnp.dot(xn, w_ref[...],
                                preferred_element_type=jnp.float32)


def _post_body(s_ref, hs_ref, deg2_ref, b_ref, o_ref):
    dinv = _dinv_blk(deg2_ref)
    o_ref[...] = dinv * (s_ref[0] + s_ref[1] + hs_ref[...]) + b_ref[...]


_GRID = (_N // _ROW_BLK,)
_SPEC_ROWS = pl.BlockSpec((_ROW_BLK, _D), lambda i: (i, 0))
_SPEC_S = pl.BlockSpec((_NC, _ROW_BLK, _D), lambda i: (0, i, 0))
_SPEC_DEG = pl.BlockSpec((_NC, _ROW_BLK, _DEG_W), lambda i: (0, i, 0))
_SPEC_W = pl.BlockSpec((_D, _D), lambda i: (0, 0))
_SPEC_B = pl.BlockSpec((1, _D), lambda i: (0, 0))
_OUT_SD = jax.ShapeDtypeStruct((_N, _D), jnp.float32)


def _pre_kernel(x, W, deg2):
    return pl.pallas_call(
        _pre_body, grid=_GRID,
        in_specs=[_SPEC_ROWS, _SPEC_W, _SPEC_DEG],
        out_specs=_SPEC_ROWS, out_shape=_OUT_SD,
    )(x, W, deg2)


def _mid_kernel(S, hs, deg2, Wn, b):
    return pl.pallas_call(
        _mid_body, grid=_GRID,
        in_specs=[_SPEC_S, _SPEC_ROWS, _SPEC_DEG, _SPEC_W, _SPEC_B],
        out_specs=_SPEC_ROWS, out_shape=_OUT_SD,
    )(S, hs, deg2, Wn, b)


def _post_kernel(S, hs, deg2, b):
    return pl.pallas_call(
        _post_body, grid=_GRID,
        in_specs=[_SPEC_S, _SPEC_ROWS, _SPEC_DEG, _SPEC_B],
        out_specs=_SPEC_ROWS, out_shape=_OUT_SD,
    )(S, hs, deg2, b)


def kernel(x, edge_index, W0, b0, W1, b1, W2, b2):
    pad = _EPAD - _E
    src = jnp.concatenate([edge_index[0], jnp.zeros((pad,), jnp.int32)])
    dst = jnp.concatenate([edge_index[1], jnp.full((pad,), _N, jnp.int32)])
    src_slab = src.reshape(_TCH, _CH)
    dst_slab = dst.reshape(_TCH, _CH)

    deg2 = _deg_kernel(dst_slab)

    hs0 = _pre_kernel(x, W0, deg2)
    S0 = _agg_kernel(hs0, src_slab, dst_slab)
    hs1 = _mid_kernel(S0, hs0, deg2, W1, b0.reshape(1, _D))
    S1 = _agg_kernel(hs1, src_slab, dst_slab)
    hs2 = _mid_kernel(S1, hs1, deg2, W2, b1.reshape(1, _D))
    S2 = _agg_kernel(hs2, src_slab, dst_slab)
    return _post_kernel(S2, hs2, deg2, b2.reshape(1, _D))
